# baseline (speedup 1.0000x reference)
import jax
import jax.numpy as jnp
from jax.experimental import pallas as pl
from jax.experimental.pallas import tpu as pltpu

NUM_CODES = 1024
DIM = 64
COMMIT_W = 0.25
TILE = 1024
ROWS = 8


def _vq_body(x_ref, w_ref, q_ref, idx_ref, loss_ref,
             w2_ref, w2x_ref, rowf_ref):
    i = pl.program_id(0)

    @pl.when(i == 0)
    def _prep():
        w = w_ref[...]
        w2_ref[...] = jnp.sum(w * w, axis=1, keepdims=True)
        w2x_ref[...] = w + w
        rowf_ref[...] = jax.lax.broadcasted_iota(
            jnp.int32, (NUM_CODES, TILE), 0).astype(jnp.float32)

    rowf = rowf_ref[...]
    part = jnp.zeros((1, 1), jnp.float32)
    for r in range(ROWS):
        xb = x_ref[r]
        x2 = jnp.sum(xb * xb, axis=0, keepdims=True)
        m2 = jax.lax.dot_general(w2x_ref[...], xb, (((1,), (0,)), ((), ())),
                                 preferred_element_type=jnp.float32)
        d = (x2 + w2_ref[...]) - m2
        dmin = jnp.min(d, axis=0, keepdims=True)
        idxf = jnp.min(jnp.where(d == dmin, rowf, jnp.float32(NUM_CODES)),
                       axis=0, keepdims=True)
        idx_ref[r] = idxf.astype(jnp.int32)
        onehot = (rowf == idxf).astype(jnp.float32)
        q = jax.lax.dot_general(w_ref[...], onehot, (((0,), (0,)), ((), ())),
                                preferred_element_type=jnp.float32)
        q_ref[r] = q
        diff = xb - q
        part = part + jnp.sum(diff * diff).reshape(1, 1)

    @pl.when(i == 0)
    def _init():
        loss_ref[...] = part

    @pl.when(i != 0)
    def _acc():
        loss_ref[...] += part


def kernel(x, W):
    B, D, T = x.shape
    N = B * T
    nb = B // ROWS

    q, idx3, loss = pl.pallas_call(
        _vq_body,
        grid=(nb,),
        in_specs=[
            pl.BlockSpec((ROWS, D, TILE), lambda i: (i, 0, 0)),
            pl.BlockSpec((NUM_CODES, D), lambda i: (0, 0)),
        ],
        out_specs=[
            pl.BlockSpec((ROWS, D, TILE), lambda i: (i, 0, 0)),
            pl.BlockSpec((ROWS, 1, TILE), lambda i: (i, 0, 0)),
            pl.BlockSpec((1, 1), lambda i: (0, 0)),
        ],
        out_shape=[
            jax.ShapeDtypeStruct((B, D, T), jnp.float32),
            jax.ShapeDtypeStruct((B, 1, T), jnp.int32),
            jax.ShapeDtypeStruct((1, 1), jnp.float32),
        ],
        scratch_shapes=[
            pltpu.VMEM((NUM_CODES, 1), jnp.float32),
            pltpu.VMEM((NUM_CODES, DIM), jnp.float32),
            pltpu.VMEM((NUM_CODES, TILE), jnp.float32),
        ],
    )(x, W)
    indices = idx3.reshape(B, T)
    commitment_loss = COMMIT_W * (loss[0, 0] / (N * D))
    return (q, indices, commitment_loss)


# speedup vs baseline: 1.0179x; 1.0179x over previous
import jax
import jax.numpy as jnp
from jax.experimental import pallas as pl
from jax.experimental.pallas import tpu as pltpu

NUM_CODES = 1024
DIM = 64
COMMIT_W = 0.25
TILE = 1024
ROWS = 4


def _vq_body(x_ref, w_ref, q_ref, idx_ref, loss_ref,
             w2_ref, w2x_ref, rowf_ref):
    i = pl.program_id(0)

    @pl.when(i == 0)
    def _prep():
        w = w_ref[...]
        w2_ref[...] = jnp.sum(w * w, axis=1, keepdims=True)
        w2x_ref[...] = w + w
        rowf_ref[...] = jax.lax.broadcasted_iota(
            jnp.int32, (NUM_CODES, TILE), 0).astype(jnp.float32)

    rowf = rowf_ref[...]
    part = jnp.zeros((1, 1), jnp.float32)
    for r in range(ROWS):
        xb = x_ref[r]
        x2 = jnp.sum(xb * xb, axis=0, keepdims=True)
        m2 = jax.lax.dot_general(w2x_ref[...], xb, (((1,), (0,)), ((), ())),
                                 preferred_element_type=jnp.float32)
        d = (x2 + w2_ref[...]) - m2
        dmin = jnp.min(d, axis=0, keepdims=True)
        idxf = jnp.min(jnp.where(d == dmin, rowf, jnp.float32(NUM_CODES)),
                       axis=0, keepdims=True)
        idx_ref[r] = idxf.astype(jnp.int32)
        onehot = (rowf == idxf).astype(jnp.float32)
        q = jax.lax.dot_general(w_ref[...], onehot, (((0,), (0,)), ((), ())),
                                preferred_element_type=jnp.float32)
        q_ref[r] = q
        diff = xb - q
        part = part + jnp.sum(diff * diff).reshape(1, 1)

    @pl.when(i == 0)
    def _init():
        loss_ref[...] = part

    @pl.when(i != 0)
    def _acc():
        loss_ref[...] += part


def kernel(x, W):
    B, D, T = x.shape
    N = B * T
    nb = B // ROWS

    q, idx3, loss = pl.pallas_call(
        _vq_body,
        grid=(nb,),
        in_specs=[
            pl.BlockSpec((ROWS, D, TILE), lambda i: (i, 0, 0)),
            pl.BlockSpec((NUM_CODES, D), lambda i: (0, 0)),
        ],
        out_specs=[
            pl.BlockSpec((ROWS, D, TILE), lambda i: (i, 0, 0)),
            pl.BlockSpec((ROWS, 1, TILE), lambda i: (i, 0, 0)),
            pl.BlockSpec((1, 1), lambda i: (0, 0)),
        ],
        out_shape=[
            jax.ShapeDtypeStruct((B, D, T), jnp.float32),
            jax.ShapeDtypeStruct((B, 1, T), jnp.int32),
            jax.ShapeDtypeStruct((1, 1), jnp.float32),
        ],
        scratch_shapes=[
            pltpu.VMEM((NUM_CODES, 1), jnp.float32),
            pltpu.VMEM((NUM_CODES, DIM), jnp.float32),
            pltpu.VMEM((NUM_CODES, TILE), jnp.float32),
        ],
    )(x, W)
    indices = idx3.reshape(B, T)
    commitment_loss = COMMIT_W * (loss[0, 0] / (N * D))
    return (q, indices, commitment_loss)
